# Initial kernel scaffold; baseline (speedup 1.0000x reference)
import jax
import jax.numpy as jnp
from jax.experimental import pallas as pl


def kernel(inputs, table):
    raise NotImplementedError("write your pallas kernel here")


# speedup vs baseline: 3.6373x; 3.6373x over previous
import functools

import jax
import jax.numpy as jnp
from jax import lax
from jax.experimental import pallas as pl
from jax.experimental.pallas import tpu as pltpu
from jax.experimental.pallas import tpu_sc as plsc

_INFO = plsc.get_sparse_core_info()
_NC = _INFO.num_cores
_NS = _INFO.num_subcores
_NW = _NC * _NS


@functools.lru_cache(maxsize=None)
def _make_emb(n_total: int, dim: int):
    assert n_total % _NW == 0
    per_w = n_total // _NW
    chunk = 512
    while per_w % chunk:
        chunk //= 2
    nchunk = per_w // chunk
    mesh = plsc.VectorSubcoreMesh(core_axis_name="c", subcore_axis_name="s")

    @functools.partial(
        pl.kernel,
        mesh=mesh,
        out_type=jax.ShapeDtypeStruct((n_total, dim), jnp.float32),
        scratch_types=[
            pltpu.VMEM((per_w,), jnp.int32),
            pltpu.VMEM((2, chunk, dim), jnp.float32),
            pltpu.SemaphoreType.DMA,
        ],
        compiler_params=pltpu.CompilerParams(use_tc_tiling_on_sc=False),
    )
    def emb(idx_hbm, table_hbm, out_hbm, idx_v, rows_v, gsem):
        wid = lax.axis_index("s") * _NC + lax.axis_index("c")
        base = wid * per_w
        pltpu.sync_copy(idx_hbm.at[pl.ds(base, per_w)], idx_v)
        for g in range(nchunk):
            buf = rows_v.at[g % 2]
            pltpu.async_copy(
                table_hbm.at[idx_v.at[pl.ds(g * chunk, chunk)]], buf, gsem
            ).wait()
            pltpu.sync_copy(buf, out_hbm.at[pl.ds(base + g * chunk, chunk)])

    return emb


def kernel(inputs, table):
    b, s = inputs.shape
    dim = table.shape[1]
    idx = inputs.reshape(-1).astype(jnp.int32)
    out = _make_emb(idx.shape[0], dim)(idx, table.astype(jnp.float32))
    return out.reshape(b, s, dim)


# speedup vs baseline: 3.6377x; 1.0001x over previous
import functools

import jax
import jax.numpy as jnp
from jax import lax
from jax.experimental import pallas as pl
from jax.experimental.pallas import tpu as pltpu
from jax.experimental.pallas import tpu_sc as plsc

_INFO = plsc.get_sparse_core_info()
_NC = _INFO.num_cores
_NS = _INFO.num_subcores
_NW = _NC * _NS


@functools.lru_cache(maxsize=None)
def _make_emb(n_total: int, dim: int):
    assert n_total % _NW == 0
    per_w = n_total // _NW
    chunk = 512
    while per_w % chunk:
        chunk //= 2
    nchunk = per_w // chunk
    nbuf = min(3, nchunk)
    mesh = plsc.VectorSubcoreMesh(core_axis_name="c", subcore_axis_name="s")

    @functools.partial(
        pl.kernel,
        mesh=mesh,
        out_type=jax.ShapeDtypeStruct((n_total, dim), jnp.float32),
        scratch_types=[
            pltpu.VMEM((per_w,), jnp.int32),
            pltpu.VMEM((nbuf, chunk, dim), jnp.float32),
            pltpu.SemaphoreType.DMA((nbuf,)),
            pltpu.SemaphoreType.DMA((nbuf,)),
        ],
        compiler_params=pltpu.CompilerParams(use_tc_tiling_on_sc=False),
    )
    def emb(idx_hbm, table_hbm, out_hbm, idx_v, rows_v, gsem, osem):
        wid = lax.axis_index("s") * _NC + lax.axis_index("c")
        base = wid * per_w

        def gather(g):
            b = g % nbuf
            return pltpu.make_async_copy(
                table_hbm.at[idx_v.at[pl.ds(g * chunk, chunk)]],
                rows_v.at[b],
                gsem.at[b],
            )

        def put(g):
            b = g % nbuf
            return pltpu.make_async_copy(
                rows_v.at[b],
                out_hbm.at[pl.ds(base + g * chunk, chunk)],
                osem.at[b],
            )

        pltpu.sync_copy(idx_hbm.at[pl.ds(base, per_w)], idx_v)
        for g in range(nbuf):
            gather(g).start()
        for g in range(nchunk):
            gather(g).wait()
            put(g).start()
            nx = g + nbuf
            if nx < nchunk:
                put(g).wait()
                gather(nx).start()
        for g in range(nchunk - nbuf, nchunk):
            if g >= 0:
                put(g).wait()

    return emb


def kernel(inputs, table):
    b, s = inputs.shape
    dim = table.shape[1]
    idx = inputs.reshape(-1).astype(jnp.int32)
    out = _make_emb(idx.shape[0], dim)(idx, table.astype(jnp.float32))
    return out.reshape(b, s, dim)


# speedup vs baseline: 4.5294x; 1.2451x over previous
import functools

import jax
import jax.numpy as jnp
from jax import lax
from jax.experimental import pallas as pl
from jax.experimental.pallas import tpu as pltpu
from jax.experimental.pallas import tpu_sc as plsc

_INFO = plsc.get_sparse_core_info()
_NC = _INFO.num_cores
_NS = _INFO.num_subcores
_NW = _NC * _NS


@functools.lru_cache(maxsize=None)
def _make_emb(n_total: int, vocab: int, dim: int):
    assert n_total % _NW == 0
    per_w = n_total // _NW
    chunk = 512
    while per_w % chunk:
        chunk //= 2
    nchunk = per_w // chunk
    nbuf = min(3, nchunk)
    mesh = plsc.VectorSubcoreMesh(core_axis_name="c", subcore_axis_name="s")

    @functools.partial(
        pl.kernel,
        mesh=mesh,
        out_type=jax.ShapeDtypeStruct((n_total, dim), jnp.float32),
        scratch_types=[
            pltpu.VMEM((per_w,), jnp.int32),
            pltpu.VMEM((nbuf, chunk, dim), jnp.float32),
            pltpu.VMEM_SHARED((vocab, dim), jnp.float32),
            pltpu.SemaphoreType.DMA((nbuf,)),
            pltpu.SemaphoreType.DMA((nbuf,)),
        ],
        compiler_params=pltpu.CompilerParams(use_tc_tiling_on_sc=False),
    )
    def emb(idx_hbm, table_hbm, out_hbm, idx_v, rows_v, table_sp, gsem, osem):
        sid = lax.axis_index("s")
        wid = sid * _NC + lax.axis_index("c")
        base = wid * per_w

        @pl.when(sid == 0)
        def _stage_table():
            pltpu.sync_copy(table_hbm, table_sp)

        def gather(g):
            b = g % nbuf
            return pltpu.make_async_copy(
                table_sp.at[idx_v.at[pl.ds(g * chunk, chunk)]],
                rows_v.at[b],
                gsem.at[b],
            )

        def put(g):
            b = g % nbuf
            return pltpu.make_async_copy(
                rows_v.at[b],
                out_hbm.at[pl.ds(base + g * chunk, chunk)],
                osem.at[b],
            )

        pltpu.sync_copy(idx_hbm.at[pl.ds(base, per_w)], idx_v)
        plsc.subcore_barrier()
        for g in range(nbuf):
            gather(g).start()
        for g in range(nchunk):
            gather(g).wait()
            put(g).start()
            nx = g + nbuf
            if nx < nchunk:
                put(g).wait()
                gather(nx).start()
        for g in range(nchunk - nbuf, nchunk):
            if g >= 0:
                put(g).wait()

    return emb


def kernel(inputs, table):
    b, s = inputs.shape
    dim = table.shape[1]
    idx = inputs.reshape(-1).astype(jnp.int32)
    out = _make_emb(idx.shape[0], table.shape[0], dim)(idx, table.astype(jnp.float32))
    return out.reshape(b, s, dim)


# speedup vs baseline: 4.8270x; 1.0657x over previous
import functools

import jax
import jax.numpy as jnp
from jax import lax
from jax.experimental import pallas as pl
from jax.experimental.pallas import tpu as pltpu
from jax.experimental.pallas import tpu_sc as plsc

_INFO = plsc.get_sparse_core_info()
_NC = _INFO.num_cores
_NS = _INFO.num_subcores
_NW = _NC * _NS


@functools.lru_cache(maxsize=None)
def _make_emb(batch: int, seq: int, vocab: int, dim: int):
    assert batch % _NW == 0
    rows_per_w = batch // _NW
    chunk_rows = 64
    while rows_per_w % chunk_rows:
        chunk_rows //= 2
    nchunk = rows_per_w // chunk_rows
    mesh = plsc.VectorSubcoreMesh(core_axis_name="c", subcore_axis_name="s")

    @functools.partial(
        pl.kernel,
        mesh=mesh,
        out_type=jax.ShapeDtypeStruct((batch, seq, dim), jnp.float32),
        scratch_types=[
            pltpu.VMEM((seq, rows_per_w), jnp.int32),
            pltpu.VMEM((seq, chunk_rows, dim), jnp.float32),
            pltpu.VMEM_SHARED((vocab, dim), jnp.float32),
            pltpu.SemaphoreType.DMA((seq,)),
            pltpu.SemaphoreType.DMA((seq,)),
        ],
        compiler_params=pltpu.CompilerParams(use_tc_tiling_on_sc=False),
    )
    def emb(idx_hbm, table_hbm, out_hbm, idx_v, rows_v, table_sp, gsem, osem):
        sid = lax.axis_index("s")
        wid = sid * _NC + lax.axis_index("c")
        row_base = wid * rows_per_w

        @pl.when(sid == 0)
        def _stage_table():
            pltpu.sync_copy(table_hbm, table_sp)

        def gather(c, s):
            return pltpu.make_async_copy(
                table_sp.at[idx_v.at[s, pl.ds(c * chunk_rows, chunk_rows)]],
                rows_v.at[s],
                gsem.at[s],
            )

        def put(c, s):
            return pltpu.make_async_copy(
                rows_v.at[s],
                out_hbm.at[pl.ds(row_base + c * chunk_rows, chunk_rows), s, :],
                osem.at[s],
            )

        pltpu.sync_copy(idx_hbm.at[:, pl.ds(row_base, rows_per_w)], idx_v)
        plsc.subcore_barrier()

        @pl.loop(0, nchunk)
        def _chunk(c):
            @pl.when(c > 0)
            def _drain():
                for s in range(seq):
                    put(c - 1, s).wait()

            for s in range(seq):
                gather(c, s).start()
            for s in range(seq):
                gather(c, s).wait()
                put(c, s).start()

        for s in range(seq):
            put(nchunk - 1, s).wait()

    return emb


def kernel(inputs, table):
    b, s = inputs.shape
    idx_t = inputs.T.astype(jnp.int32)
    return _make_emb(b, s, table.shape[0], table.shape[1])(
        idx_t, table.astype(jnp.float32)
    )
